# baseline (speedup 1.0000x reference)
import functools

import jax
import jax.numpy as jnp
from jax import lax
from jax.experimental import pallas as pl
from jax.experimental.pallas import tpu as pltpu
from jax.experimental.pallas import tpu_sc as plsc

N = 10000
E = 320000
D = 128
H = D // 2
NUM_LAYERS = 4

NC = 2
NS = 16
C = 128
CW = 160
E_PAD = NS * CW * C
RPT = 624
REM = N - RPT * NS
HB = H // 16

_mesh = plsc.VectorSubcoreMesh(
    core_axis_name="c", subcore_axis_name="s", num_cores=NC, num_subcores=NS
)


def _prop_body(x_hbm, col_hbm, row_hbm, w_hbm, zero_hbm, out_hbm,
               colv, rowv, wv, rowsv,
               gs0, gs1, gs2, ss0, ss1, ss2, acc):
    cid = lax.axis_index("c")
    sid = lax.axis_index("s")
    gsem = (gs0, gs1, gs2)
    ssem = (ss0, ss1, ss2)

    for k in range(3):
        sl = pl.ds(sid * RPT + k * (RPT // 3), RPT // 3)
        pltpu.sync_copy(zero_hbm.at[sl], acc.at[sl])

    @pl.when(sid == 0)
    def _():
        sl = pl.ds(RPT * NS, REM)
        pltpu.sync_copy(zero_hbm.at[sl], acc.at[sl])

    r0 = sid * CW
    pltpu.sync_copy(col_hbm.at[pl.ds(r0, CW)], colv)
    pltpu.sync_copy(row_hbm.at[pl.ds(r0, CW)], rowv)
    pltpu.sync_copy(w_hbm.at[pl.ds(r0, CW)], wv)

    cid_n = cid * N

    def cadj(r, carry):
        for dblk in range(C // 16):
            sl = pl.ds(dblk * 16, 16)
            colv[r, sl] = colv[r, sl] + cid_n
        return carry

    lax.fori_loop(0, CW, cadj, None)

    plsc.subcore_barrier()

    def issue_gather(k, b):
        pltpu.async_copy(x_hbm.at[colv.at[k]], rowsv.at[b], gsem[b])

    def wait_gather(k, b):
        pltpu.make_async_copy(x_hbm.at[colv.at[k]], rowsv.at[b], gsem[b]).wait()

    def issue_scatter(k, b):
        pltpu.async_copy(rowsv.at[b], acc.at[rowv.at[k]], ssem[b], add=True)

    def wait_scatter(k, b):
        pltpu.make_async_copy(rowsv.at[b], acc.at[rowv.at[k]], ssem[b]).wait()

    def scale_chunk(k, b):
        def sg(g, c2):
            wvec = wv[k, pl.ds(g * 16, 16)]
            for lane in range(16):
                we = wvec[lane]
                e = g * 16 + lane
                for dblk in range(HB):
                    sl = pl.ds(dblk * 16, 16)
                    rowsv[b, e, sl] = rowsv[b, e, sl] * we
            return c2

        lax.fori_loop(0, C // 16, sg, None)

    def maybe_gather(k, bn):
        @pl.when(k < CW - 2)
        def _():
            issue_gather(k + 2, bn)

    issue_gather(0, 0)
    issue_gather(1, 1)
    wait_gather(0, 0)
    scale_chunk(0, 0)
    issue_scatter(0, 0)
    issue_gather(2, 2)

    def body3(j, carry):
        for i in range(3):
            k = 1 + j * 3 + i
            b = (1 + i) % 3
            bn = (b + 2) % 3
            wait_gather(k, b)
            scale_chunk(k, b)
            issue_scatter(k, b)
            wait_scatter(k - 1, bn)
            maybe_gather(k, bn)
        return carry

    lax.fori_loop(0, (CW - 1) // 3, body3, None)
    wait_scatter(CW - 1, (CW - 1) % 3)

    plsc.subcore_barrier()
    r0 = sid * RPT
    pltpu.sync_copy(acc.at[pl.ds(r0, RPT)],
                    out_hbm.at[pl.ds(cid * N + r0, RPT)])

    @pl.when(sid == 0)
    def _():
        pltpu.sync_copy(acc.at[pl.ds(RPT * NS, REM)],
                        out_hbm.at[pl.ds(cid * N + RPT * NS, REM)])


_sc_propagate = pl.kernel(
    _prop_body,
    out_type=jax.ShapeDtypeStruct((NC * N, H), jnp.float32),
    mesh=_mesh,
    scratch_types=[
        pltpu.VMEM((CW, C), jnp.int32),
        pltpu.VMEM((CW, C), jnp.int32),
        pltpu.VMEM((CW, C), jnp.float32),
        pltpu.VMEM((3, C, H), jnp.float32),
        pltpu.SemaphoreType.DMA,
        pltpu.SemaphoreType.DMA,
        pltpu.SemaphoreType.DMA,
        pltpu.SemaphoreType.DMA,
        pltpu.SemaphoreType.DMA,
        pltpu.SemaphoreType.DMA,
        pltpu.VMEM_SHARED((N, H), jnp.float32),
    ],
    compiler_params=pltpu.CompilerParams(use_tc_tiling_on_sc=False),
)


def _acc_body(x_ref, acc_ref, accn_ref, *, scale):
    accn_ref[...] = (acc_ref[...] + x_ref[...]) * scale


def _accum(x2, acc, scale):
    bn = 400
    grid = (NC * N) // bn
    bs = pl.BlockSpec((bn, H), lambda i: (i, 0))
    return pl.pallas_call(
        functools.partial(_acc_body, scale=scale),
        grid=(grid,),
        in_specs=[bs, bs],
        out_specs=bs,
        out_shape=jax.ShapeDtypeStruct((NC * N, H), jnp.float32),
    )(x2, acc)


def kernel(embeds, edge_index, edge_weight):
    row = edge_index[0]
    col = edge_index[1]
    pad = E_PAD - E
    col2d = jnp.concatenate(
        [col, jnp.zeros((pad,), jnp.int32)]).reshape(NS * CW, C)
    row2d = jnp.concatenate(
        [row, jnp.zeros((pad,), jnp.int32)]).reshape(NS * CW, C)
    w2d = jnp.concatenate(
        [edge_weight, jnp.zeros((pad,), jnp.float32)]).reshape(NS * CW, C)
    zeros = jnp.zeros((N, H), jnp.float32)
    x2 = jnp.concatenate([embeds[:, :H], embeds[:, H:]], axis=0)
    acc = x2
    for layer in range(NUM_LAYERS):
        x2 = _sc_propagate(x2, col2d, row2d, w2d, zeros)
        scale = 1.0 if layer < NUM_LAYERS - 1 else 1.0 / (NUM_LAYERS + 1)
        acc = _accum(x2, acc, scale)
    return jnp.concatenate([acc[:N], acc[N:]], axis=1)


# speedup vs baseline: 1.1613x; 1.1613x over previous
import functools

import jax
import jax.numpy as jnp
from jax import lax
from jax.experimental import pallas as pl
from jax.experimental.pallas import tpu as pltpu
from jax.experimental.pallas import tpu_sc as plsc

N = 10000
E = 320000
D = 128
H = D // 2
NUM_LAYERS = 4

NC = 2
NS = 16
C = 128
CW = 160
G = 40
NG = CW // G
R = 5
E_PAD = NS * CW * C
RPT = 624
REM = N - RPT * NS
HB = H // 16

_mesh = plsc.VectorSubcoreMesh(
    core_axis_name="c", subcore_axis_name="s", num_cores=NC, num_subcores=NS
)


def _prop_body(x_hbm, col_hbm, row_hbm, w_hbm, zero_hbm, out_hbm,
               colv0, colv1, rowv0, rowv1, wv0, wv1, rowsv,
               gs0, gs1, gs2, gs3, gs4, ss0, ss1, ss2, ss3, ss4,
               is0, is1, acc):
    cid = lax.axis_index("c")
    sid = lax.axis_index("s")
    colb = (colv0, colv1)
    rowb = (rowv0, rowv1)
    wb = (wv0, wv1)
    gsem = (gs0, gs1, gs2, gs3, gs4)
    ssem = (ss0, ss1, ss2, ss3, ss4)
    isem = (is0, is1)

    xsrc = x_hbm.at[pl.ds(cid * N, N)]

    for k in range(3):
        sl = pl.ds(sid * RPT + k * (RPT // 3), RPT // 3)
        pltpu.sync_copy(zero_hbm.at[sl], acc.at[sl])

    @pl.when(sid == 0)
    def _():
        sl = pl.ds(RPT * NS, REM)
        pltpu.sync_copy(zero_hbm.at[sl], acc.at[sl])

    r0 = sid * CW
    pltpu.sync_copy(col_hbm.at[pl.ds(r0, G)], colv0)
    pltpu.sync_copy(row_hbm.at[pl.ds(r0, G)], rowv0)
    pltpu.sync_copy(w_hbm.at[pl.ds(r0, G)], wv0)

    plsc.subcore_barrier()

    def issue_gather(p2, kk, b):
        pltpu.async_copy(xsrc.at[colb[p2].at[kk]], rowsv.at[b], gsem[b])

    def wait_gather(b):
        pltpu.make_async_copy(xsrc.at[colv0.at[0]], rowsv.at[b],
                              gsem[b]).wait()

    def issue_scatter(p2, kk, b):
        pltpu.async_copy(rowsv.at[b], acc.at[rowb[p2].at[kk]], ssem[b],
                         add=True)

    def wait_scatter(b):
        pltpu.make_async_copy(rowsv.at[b], acc.at[rowv0.at[0]],
                              ssem[b]).wait()

    def scale_chunk(p2, kk, b):
        wvp = wb[p2]

        def sg(g2, c2):
            wvec = wvp[kk, pl.ds(g2 * 16, 16)]
            for lane in range(16):
                we = wvec[lane]
                e = g2 * 16 + lane
                for dblk in range(HB):
                    sl = pl.ds(dblk * 16, 16)
                    rowsv[b, e, sl] = rowsv[b, e, sl] * we
            return c2

        lax.fori_loop(0, C // 16, sg, None)

    for b in range(R - 1):
        issue_gather(0, b, b)

    def chunk_step(p2, c, k, b):
        wait_gather(b)
        scale_chunk(p2, c, b)
        issue_scatter(p2, c, b)

        @pl.when(k > 0)
        def _():
            wait_scatter((b + R - 1) % R)

    def group_body(p, g):
        base = g * G
        nxt = p ^ 1

        @pl.when(g < NG - 1)
        def _():
            gr = r0 + (g + 1) * G
            pltpu.async_copy(col_hbm.at[pl.ds(gr, G)], colb[nxt], isem[nxt])
            pltpu.async_copy(row_hbm.at[pl.ds(gr, G)], rowb[nxt], isem[nxt])
            pltpu.async_copy(w_hbm.at[pl.ds(gr, G)], wb[nxt], isem[nxt])

        def octet(o, carry):
            for i in range(R):
                c = o * R + i
                chunk_step(p, c, base + c, i)
                issue_gather(p, c + R - 1, (i + R - 1) % R)
            return carry

        lax.fori_loop(0, G // R - 1, octet, None)

        for i in range(R):
            c = G - R + i
            chunk_step(p, c, base + c, i)
            if i == 0:
                issue_gather(p, G - 1, (i + R - 1) % R)
            else:
                @pl.when(g < NG - 1)
                def _(i=i):
                    if i == 1:
                        pltpu.make_async_copy(
                            col_hbm.at[pl.ds(r0, G)], colb[nxt],
                            isem[nxt]).wait()
                        pltpu.make_async_copy(
                            row_hbm.at[pl.ds(r0, G)], rowb[nxt],
                            isem[nxt]).wait()
                        pltpu.make_async_copy(
                            w_hbm.at[pl.ds(r0, G)], wb[nxt],
                            isem[nxt]).wait()
                    issue_gather(nxt, i - 1, (i + R - 1) % R)

    def pair_body(pair, carry):
        group_body(0, pair * 2)
        group_body(1, pair * 2 + 1)
        return carry

    lax.fori_loop(0, NG // 2, pair_body, None)
    wait_scatter((CW - 1) % R)

    plsc.subcore_barrier()
    r1 = sid * RPT
    pltpu.sync_copy(acc.at[pl.ds(r1, RPT)],
                    out_hbm.at[pl.ds(cid * N + r1, RPT)])

    @pl.when(sid == 0)
    def _():
        pltpu.sync_copy(acc.at[pl.ds(RPT * NS, REM)],
                        out_hbm.at[pl.ds(cid * N + RPT * NS, REM)])


_sc_propagate = pl.kernel(
    _prop_body,
    out_type=jax.ShapeDtypeStruct((NC * N, H), jnp.float32),
    mesh=_mesh,
    scratch_types=[
        pltpu.VMEM((G, C), jnp.int32),
        pltpu.VMEM((G, C), jnp.int32),
        pltpu.VMEM((G, C), jnp.int32),
        pltpu.VMEM((G, C), jnp.int32),
        pltpu.VMEM((G, C), jnp.float32),
        pltpu.VMEM((G, C), jnp.float32),
        pltpu.VMEM((R, C, H), jnp.float32),
        pltpu.SemaphoreType.DMA,
        pltpu.SemaphoreType.DMA,
        pltpu.SemaphoreType.DMA,
        pltpu.SemaphoreType.DMA,
        pltpu.SemaphoreType.DMA,
        pltpu.SemaphoreType.DMA,
        pltpu.SemaphoreType.DMA,
        pltpu.SemaphoreType.DMA,
        pltpu.SemaphoreType.DMA,
        pltpu.SemaphoreType.DMA,
        pltpu.SemaphoreType.DMA,
        pltpu.SemaphoreType.DMA,
        pltpu.VMEM_SHARED((N, H), jnp.float32),
    ],
    compiler_params=pltpu.CompilerParams(use_tc_tiling_on_sc=False),
)


def _acc_body(x_ref, acc_ref, accn_ref, *, scale):
    accn_ref[...] = (acc_ref[...] + x_ref[...]) * scale


def _accum(x2, acc, scale):
    bn = 400
    grid = (NC * N) // bn
    bs = pl.BlockSpec((bn, H), lambda i: (i, 0))
    return pl.pallas_call(
        functools.partial(_acc_body, scale=scale),
        grid=(grid,),
        in_specs=[bs, bs],
        out_specs=bs,
        out_shape=jax.ShapeDtypeStruct((NC * N, H), jnp.float32),
    )(x2, acc)


def kernel(embeds, edge_index, edge_weight):
    row = edge_index[0]
    col = edge_index[1]
    pad = E_PAD - E
    col2d = jnp.concatenate(
        [col, jnp.zeros((pad,), jnp.int32)]).reshape(NS * CW, C)
    row2d = jnp.concatenate(
        [row, jnp.zeros((pad,), jnp.int32)]).reshape(NS * CW, C)
    w2d = jnp.concatenate(
        [edge_weight, jnp.zeros((pad,), jnp.float32)]).reshape(NS * CW, C)
    zeros = jnp.zeros((N, H), jnp.float32)
    x2 = jnp.concatenate([embeds[:, :H], embeds[:, H:]], axis=0)
    acc = x2
    for layer in range(NUM_LAYERS):
        x2 = _sc_propagate(x2, col2d, row2d, w2d, zeros)
        scale = 1.0 if layer < NUM_LAYERS - 1 else 1.0 / (NUM_LAYERS + 1)
        acc = _accum(x2, acc, scale)
    return jnp.concatenate([acc[:N], acc[N:]], axis=1)
